# baseline (speedup 1.0000x reference)
import functools

import jax
import jax.numpy as jnp
from jax import lax
from jax.experimental import pallas as pl
from jax.experimental.pallas import tpu as pltpu
from jax.experimental.pallas import tpu_sc as plsc

P = 1024
O = 512
VOCAB = 200
RPAD = 256
R = 512
NC, NS = 2, 16
NW = NC * NS
CH = 32


def _precompute_body(e0_ref, e1_ref, w1a_ref, w1b_ref, b1_ref, tt_ref):
    t0 = jnp.dot(e0_ref[...], w1a_ref[...], preferred_element_type=jnp.float32,
                 precision=jax.lax.Precision.HIGHEST)
    t1 = jnp.dot(e1_ref[...], w1b_ref[...], preferred_element_type=jnp.float32,
                 precision=jax.lax.Precision.HIGHEST)
    tt_ref[0:RPAD, :] = t0.astype(jnp.bfloat16)
    tt_ref[RPAD:2 * RPAD, :] = (t1 + b1_ref[...]).astype(jnp.bfloat16)


def _consume_body(g0_ref, g1_ref, w2e_ref, w2o_ref, b2_ref, out_ref):
    a = g0_ref[...]
    b = g1_ref[...]
    even = (jax.lax.bitcast_convert_type(a << 16, jnp.float32)
            + jax.lax.bitcast_convert_type(b << 16, jnp.float32))
    hi_mask = jnp.int32(-65536)
    odd = (jax.lax.bitcast_convert_type(a & hi_mask, jnp.float32)
           + jax.lax.bitcast_convert_type(b & hi_mask, jnp.float32))
    he = jnp.maximum(even, 0.0).astype(jnp.bfloat16)
    ho = jnp.maximum(odd, 0.0).astype(jnp.bfloat16)
    out_ref[...] = (jnp.dot(he, w2e_ref[...], preferred_element_type=jnp.float32)
                    + jnp.dot(ho, w2o_ref[...], preferred_element_type=jnp.float32)
                    + b2_ref[...])


def _sc_gather(tt_hbm, i0_hbm, i1_hbm, g0_hbm, g1_hbm,
               idx_a, idx_b, buf_a, buf_b, sem_a, sem_b):
    wid = lax.axis_index("s") * NC + lax.axis_index("c")
    base = wid * (CH * 16)
    idxs = (idx_a, idx_b)
    bufs = (buf_a, buf_b)
    sems = (sem_a, sem_b)

    def start(c):
        s = c % 2
        pltpu.sync_copy(i0_hbm.at[pl.ds(base + c * CH, CH)],
                        idxs[s].at[pl.ds(0, CH)])
        pltpu.sync_copy(i1_hbm.at[pl.ds(base + c * CH, CH)],
                        idxs[s].at[pl.ds(CH, CH)])
        return pltpu.async_copy(tt_hbm.at[idxs[s]], bufs[s], sems[s])

    copies = {0: start(0)}
    for c in range(16):
        if c + 1 < 16:
            copies[c + 1] = start(c + 1)
        copies.pop(c).wait()
        s = c % 2
        pltpu.sync_copy(bufs[s].at[pl.ds(0, CH)],
                        g0_hbm.at[pl.ds(base + c * CH, CH)])
        pltpu.sync_copy(bufs[s].at[pl.ds(CH, CH)],
                        g1_hbm.at[pl.ds(base + c * CH, CH)])


def kernel(x, E0, E1, W1, b1, W2, b2):
    B = x.shape[0]
    nsteps = B // R

    e0p = jnp.pad(E0, ((0, RPAD - VOCAB), (0, 0)))
    e1p = jnp.pad(E1, ((0, RPAD - VOCAB), (0, 0)))

    tt = pl.pallas_call(
        _precompute_body,
        out_shape=jax.ShapeDtypeStruct((2 * RPAD, P), jnp.bfloat16),
    )(e0p, e1p, W1[:P], W1[P:], b1.reshape(1, P))

    i0 = x[:, 0].astype(jnp.int32)
    i1 = x[:, 1].astype(jnp.int32) + RPAD

    tt_i32 = jax.lax.bitcast_convert_type(
        tt.reshape(2 * RPAD, P // 2, 2), jnp.int32)

    mesh = plsc.VectorSubcoreMesh(core_axis_name="c", subcore_axis_name="s",
                                  num_cores=NC, num_subcores=NS)
    g0, g1 = pl.kernel(
        _sc_gather,
        out_type=[jax.ShapeDtypeStruct((B, P // 2), jnp.int32),
                  jax.ShapeDtypeStruct((B, P // 2), jnp.int32)],
        mesh=mesh,
        scratch_types=[
            pltpu.VMEM((2 * CH,), jnp.int32),
            pltpu.VMEM((2 * CH,), jnp.int32),
            pltpu.VMEM((2 * CH, P // 2), jnp.int32),
            pltpu.VMEM((2 * CH, P // 2), jnp.int32),
            pltpu.SemaphoreType.DMA,
            pltpu.SemaphoreType.DMA,
        ],
    )(tt_i32, i0, i1)

    w2e = W2[0::2].astype(jnp.bfloat16)
    w2o = W2[1::2].astype(jnp.bfloat16)

    out = pl.pallas_call(
        _consume_body,
        grid=(nsteps,),
        in_specs=[
            pl.BlockSpec((R, P // 2), lambda i: (i, 0)),
            pl.BlockSpec((R, P // 2), lambda i: (i, 0)),
            pl.BlockSpec((P // 2, O), lambda i: (0, 0)),
            pl.BlockSpec((P // 2, O), lambda i: (0, 0)),
            pl.BlockSpec((1, O), lambda i: (0, 0)),
        ],
        out_specs=pl.BlockSpec((R, O), lambda i: (i, 0)),
        out_shape=jax.ShapeDtypeStruct((B, O), jnp.float32),
    )(g0, g1, w2e, w2o, b2.reshape(1, O))
    return out


# speedup vs baseline: 1.0655x; 1.0655x over previous
import functools

import jax
import jax.numpy as jnp
from jax import lax
from jax.experimental import pallas as pl
from jax.experimental.pallas import tpu as pltpu
from jax.experimental.pallas import tpu_sc as plsc

P = 1024
O = 512
VOCAB = 200
RPAD = 256
R = 512
NC, NS = 2, 16
NW = NC * NS
CH = 32


def _precompute_body(e0_ref, e1_ref, w1a_ref, w1b_ref, b1_ref, tt_ref):
    w1a = w1a_ref[...].astype(jnp.bfloat16)
    w1b = w1b_ref[...].astype(jnp.bfloat16)
    t0 = jnp.dot(e0_ref[...].astype(jnp.bfloat16), w1a,
                 preferred_element_type=jnp.float32)
    t1 = jnp.dot(e1_ref[...].astype(jnp.bfloat16), w1b,
                 preferred_element_type=jnp.float32)
    tt_ref[0:VOCAB, :] = t0.astype(jnp.bfloat16)
    tt_ref[RPAD:RPAD + VOCAB, :] = (t1 + b1_ref[...]).astype(jnp.bfloat16)


def _consume_body(g0_ref, g1_ref, w2p_ref, b2_ref, out_ref):
    a = g0_ref[...]
    b = g1_ref[...]
    even = (jax.lax.bitcast_convert_type(a << 16, jnp.float32)
            + jax.lax.bitcast_convert_type(b << 16, jnp.float32))
    hi_mask = jnp.int32(-65536)
    odd = (jax.lax.bitcast_convert_type(a & hi_mask, jnp.float32)
           + jax.lax.bitcast_convert_type(b & hi_mask, jnp.float32))
    h = jnp.concatenate(
        [jnp.maximum(even, 0.0).astype(jnp.bfloat16),
         jnp.maximum(odd, 0.0).astype(jnp.bfloat16)], axis=1)
    out_ref[...] = (jnp.dot(h, w2p_ref[...], preferred_element_type=jnp.float32)
                    + b2_ref[...])


def _sc_gather(tt_hbm, i0_hbm, i1_hbm, g0_hbm, g1_hbm,
               idx_a, idx_b, buf_a, buf_b, sem_a, sem_b):
    wid = lax.axis_index("s") * NC + lax.axis_index("c")
    base = wid * (CH * 16)
    idxs = (idx_a, idx_b)
    bufs = (buf_a, buf_b)
    sems = (sem_a, sem_b)

    def start(c):
        s = c % 2
        pltpu.sync_copy(i0_hbm.at[pl.ds(base + c * CH, CH)],
                        idxs[s].at[pl.ds(0, CH)])
        pltpu.sync_copy(i1_hbm.at[pl.ds(base + c * CH, CH)],
                        idxs[s].at[pl.ds(CH, CH)])
        return pltpu.async_copy(tt_hbm.at[idxs[s]], bufs[s], sems[s])

    copies = {0: start(0)}
    for c in range(16):
        if c + 1 < 16:
            copies[c + 1] = start(c + 1)
        copies.pop(c).wait()
        s = c % 2
        pltpu.sync_copy(bufs[s].at[pl.ds(0, CH)],
                        g0_hbm.at[pl.ds(base + c * CH, CH)])
        pltpu.sync_copy(bufs[s].at[pl.ds(CH, CH)],
                        g1_hbm.at[pl.ds(base + c * CH, CH)])


def kernel(x, E0, E1, W1, b1, W2, b2):
    B = x.shape[0]
    nsteps = B // R

    tt = pl.pallas_call(
        _precompute_body,
        out_shape=jax.ShapeDtypeStruct((2 * RPAD, P), jnp.bfloat16),
    )(E0, E1, W1[:P], W1[P:], b1.reshape(1, P))

    i0 = x[:, 0].astype(jnp.int32)
    i1 = x[:, 1].astype(jnp.int32) + RPAD

    tt_i32 = jax.lax.bitcast_convert_type(
        tt.reshape(2 * RPAD, P // 2, 2), jnp.int32)

    mesh = plsc.VectorSubcoreMesh(core_axis_name="c", subcore_axis_name="s",
                                  num_cores=NC, num_subcores=NS)
    g0, g1 = pl.kernel(
        _sc_gather,
        out_type=[jax.ShapeDtypeStruct((B, P // 2), jnp.int32),
                  jax.ShapeDtypeStruct((B, P // 2), jnp.int32)],
        mesh=mesh,
        scratch_types=[
            pltpu.VMEM((2 * CH,), jnp.int32),
            pltpu.VMEM((2 * CH,), jnp.int32),
            pltpu.VMEM((2 * CH, P // 2), jnp.int32),
            pltpu.VMEM((2 * CH, P // 2), jnp.int32),
            pltpu.SemaphoreType.DMA,
            pltpu.SemaphoreType.DMA,
        ],
    )(tt_i32, i0, i1)

    w2p = jnp.concatenate([W2[0::2], W2[1::2]], axis=0).astype(jnp.bfloat16)

    out = pl.pallas_call(
        _consume_body,
        grid=(nsteps,),
        in_specs=[
            pl.BlockSpec((R, P // 2), lambda i: (i, 0)),
            pl.BlockSpec((R, P // 2), lambda i: (i, 0)),
            pl.BlockSpec((P, O), lambda i: (0, 0)),
            pl.BlockSpec((1, O), lambda i: (0, 0)),
        ],
        out_specs=pl.BlockSpec((R, O), lambda i: (i, 0)),
        out_shape=jax.ShapeDtypeStruct((B, O), jnp.float32),
    )(g0, g1, w2p, b2.reshape(1, O))
    return out


# speedup vs baseline: 1.1897x; 1.1166x over previous
import functools

import jax
import jax.numpy as jnp
from jax import lax
from jax.experimental import pallas as pl
from jax.experimental.pallas import tpu as pltpu
from jax.experimental.pallas import tpu_sc as plsc

P = 1024
O = 512
VOCAB = 200
RPAD = 256
R = 512
NC, NS = 2, 16
NW = NC * NS
CH = 32
B_SC = 8192


def _precompute_body(e0_ref, e1_ref, w1a_ref, w1b_ref, b1_ref, tt_ref):
    w1a = w1a_ref[...].astype(jnp.bfloat16)
    w1b = w1b_ref[...].astype(jnp.bfloat16)
    t0 = jnp.dot(e0_ref[...].astype(jnp.bfloat16), w1a,
                 preferred_element_type=jnp.float32)
    t1 = jnp.dot(e1_ref[...].astype(jnp.bfloat16), w1b,
                 preferred_element_type=jnp.float32)
    tt_ref[0:VOCAB, :] = t0.astype(jnp.bfloat16)
    tt_ref[RPAD:RPAD + VOCAB, :] = (t1 + b1_ref[...]).astype(jnp.bfloat16)


def _onehot_body(i0_ref, i1_ref, tt_ref, w2_ref, b2_ref, out_ref):
    iv = i0_ref[0]
    jv = i1_ref[0]
    col = jax.lax.broadcasted_iota(jnp.int32, (R, 2 * RPAD), 1)
    oh = ((col == iv) | (col == jv)).astype(jnp.bfloat16)
    g = jnp.dot(oh, tt_ref[...], preferred_element_type=jnp.float32)
    h = jnp.maximum(g, 0.0).astype(jnp.bfloat16)
    out_ref[...] = jnp.dot(h, w2_ref[...], preferred_element_type=jnp.float32) + b2_ref[...]


def _consume_body(g0_ref, g1_ref, w2p_ref, b2_ref, out_ref):
    a = g0_ref[...]
    b = g1_ref[...]
    even = (jax.lax.bitcast_convert_type(a << 16, jnp.float32)
            + jax.lax.bitcast_convert_type(b << 16, jnp.float32))
    hi_mask = jnp.int32(-65536)
    odd = (jax.lax.bitcast_convert_type(a & hi_mask, jnp.float32)
           + jax.lax.bitcast_convert_type(b & hi_mask, jnp.float32))
    h = jnp.concatenate(
        [jnp.maximum(even, 0.0).astype(jnp.bfloat16),
         jnp.maximum(odd, 0.0).astype(jnp.bfloat16)], axis=1)
    out_ref[...] = (jnp.dot(h, w2p_ref[...], preferred_element_type=jnp.float32)
                    + b2_ref[...])


def _sc_gather(tt_hbm, i0_hbm, i1_hbm, g0_hbm, g1_hbm,
               idx_a, idx_b, buf_a, buf_b, sem_a, sem_b):
    nchunk = B_SC // NW // CH
    wid = lax.axis_index("s") * NC + lax.axis_index("c")
    base = wid * (CH * nchunk)
    idxs = (idx_a, idx_b)
    bufs = (buf_a, buf_b)
    sems = (sem_a, sem_b)

    def start(c):
        s = c % 2
        pltpu.sync_copy(i0_hbm.at[pl.ds(base + c * CH, CH)],
                        idxs[s].at[pl.ds(0, CH)])
        pltpu.sync_copy(i1_hbm.at[pl.ds(base + c * CH, CH)],
                        idxs[s].at[pl.ds(CH, CH)])
        return pltpu.async_copy(tt_hbm.at[idxs[s]], bufs[s], sems[s])

    copies = {0: start(0)}
    for c in range(nchunk):
        if c + 1 < nchunk:
            copies[c + 1] = start(c + 1)
        copies.pop(c).wait()
        s = c % 2
        pltpu.sync_copy(bufs[s].at[pl.ds(0, CH)],
                        g0_hbm.at[pl.ds(base + c * CH, CH)])
        pltpu.sync_copy(bufs[s].at[pl.ds(CH, CH)],
                        g1_hbm.at[pl.ds(base + c * CH, CH)])


def kernel(x, E0, E1, W1, b1, W2, b2):
    B = x.shape[0]
    b_tc = B - B_SC

    tt = pl.pallas_call(
        _precompute_body,
        out_shape=jax.ShapeDtypeStruct((2 * RPAD, P), jnp.bfloat16),
    )(E0, E1, W1[:P], W1[P:], b1.reshape(1, P))

    i0 = x[:, 0].astype(jnp.int32)
    i1 = x[:, 1].astype(jnp.int32) + RPAD

    tt_i32 = jax.lax.bitcast_convert_type(
        tt.reshape(2 * RPAD, P // 2, 2), jnp.int32)

    mesh = plsc.VectorSubcoreMesh(core_axis_name="c", subcore_axis_name="s",
                                  num_cores=NC, num_subcores=NS)
    g0, g1 = pl.kernel(
        _sc_gather,
        out_type=[jax.ShapeDtypeStruct((B_SC, P // 2), jnp.int32),
                  jax.ShapeDtypeStruct((B_SC, P // 2), jnp.int32)],
        mesh=mesh,
        scratch_types=[
            pltpu.VMEM((2 * CH,), jnp.int32),
            pltpu.VMEM((2 * CH,), jnp.int32),
            pltpu.VMEM((2 * CH, P // 2), jnp.int32),
            pltpu.VMEM((2 * CH, P // 2), jnp.int32),
            pltpu.SemaphoreType.DMA,
            pltpu.SemaphoreType.DMA,
        ],
    )(tt_i32, i0[b_tc:], i1[b_tc:])

    w2b = W2.astype(jnp.bfloat16)
    w2p = jnp.concatenate([W2[0::2], W2[1::2]], axis=0).astype(jnp.bfloat16)
    b2r = b2.reshape(1, O)

    n_tc = b_tc // R
    out_tc = pl.pallas_call(
        _onehot_body,
        grid=(n_tc,),
        in_specs=[
            pl.BlockSpec((1, R, 1), lambda i: (i, 0, 0)),
            pl.BlockSpec((1, R, 1), lambda i: (i, 0, 0)),
            pl.BlockSpec((2 * RPAD, P), lambda i: (0, 0)),
            pl.BlockSpec((P, O), lambda i: (0, 0)),
            pl.BlockSpec((1, O), lambda i: (0, 0)),
        ],
        out_specs=pl.BlockSpec((R, O), lambda i: (i, 0)),
        out_shape=jax.ShapeDtypeStruct((b_tc, O), jnp.float32),
    )(i0[:b_tc].reshape(n_tc, R, 1), i1[:b_tc].reshape(n_tc, R, 1),
      tt, w2b, b2r)

    n_sc = B_SC // R
    out_sc = pl.pallas_call(
        _consume_body,
        grid=(n_sc,),
        in_specs=[
            pl.BlockSpec((R, P // 2), lambda i: (i, 0)),
            pl.BlockSpec((R, P // 2), lambda i: (i, 0)),
            pl.BlockSpec((P, O), lambda i: (0, 0)),
            pl.BlockSpec((1, O), lambda i: (0, 0)),
        ],
        out_specs=pl.BlockSpec((R, O), lambda i: (i, 0)),
        out_shape=jax.ShapeDtypeStruct((B_SC, O), jnp.float32),
    )(g0, g1, w2p, b2r)

    return jnp.concatenate([out_tc, out_sc], axis=0)
